# baseline (speedup 1.0000x reference)
import functools

import jax
import jax.numpy as jnp
from jax import lax
from jax.experimental import pallas as pl
from jax.experimental.pallas import tpu as pltpu
from jax.experimental.pallas import tpu_sc as plsc

N = 10000
E = 320000
NC = 2
NS = 16
NW = NC * NS
EPW = E // NW
RPS = 624
TAIL = N - NS * RPS
DEG_W = 8
DBLK = 1000
DNB = EPW // DBLK
_EDGE_BLK = {64: 200, 32: 400, 16: 1000}

_MESH = plsc.VectorSubcoreMesh(core_axis_name="c", subcore_axis_name="s")
_SC_PARAMS = pltpu.CompilerParams(use_tc_tiling_on_sc=False)


def _striped(s, mk):
    mk(s * RPS, RPS)

    @pl.when(s == NS - 1)
    def _():
        mk(NS * RPS, TAIL)


def _blk(base, j, blk):
    return pl.ds(pl.multiple_of(base + j * blk, 8), blk)


def _deg_body(e1_hbm, e2_hbm, e3_hbm, ones_hbm, zeros_hbm, out_hbm,
              ones_v, col_a, col_b, acc0, acc1, acc2, sem_a, sem_b):
    c = lax.axis_index("c")
    s = lax.axis_index("s")
    wid = c * NS + s
    ebase = wid * EPW
    for acc in (acc0, acc1, acc2):
        _striped(s, lambda o, n, acc=acc: pltpu.sync_copy(
            zeros_hbm.at[pl.ds(o, n)], acc.at[pl.ds(o, n)]))
    pltpu.sync_copy(ones_hbm, ones_v)
    plsc.subcore_barrier()

    for e_hbm, acc in ((e1_hbm, acc0), (e2_hbm, acc1), (e3_hbm, acc2)):
        def start(j, buf, sem, e_hbm=e_hbm):
            pltpu.async_copy(e_hbm.at[1, _blk(ebase, j, DBLK)], buf, sem)

        def proc(j, buf, sem, obuf, osem, e_hbm=e_hbm, acc=acc,
                 start=start):
            @pl.when(j + 1 < DNB)
            def _():
                start(j + 1, obuf, osem)

            pltpu.make_async_copy(
                e_hbm.at[1, _blk(ebase, j, DBLK)], buf, sem).wait()
            pltpu.sync_copy(ones_v, acc.at[buf], add=True)

        start(0, col_a, sem_a)

        def body(j, carry, proc=proc):
            @pl.when(lax.rem(j, 2) == 0)
            def _():
                proc(j, col_a, sem_a, col_b, sem_b)

            @pl.when(lax.rem(j, 2) == 1)
            def _():
                proc(j, col_b, sem_b, col_a, sem_a)

            return carry

        lax.fori_loop(0, DNB, body, 0)
    plsc.subcore_barrier()
    for cc in range(NC):
        @pl.when(c == cc)
        def _(cc=cc):
            for i, acc in enumerate((acc0, acc1, acc2)):
                co = 64 * cc + 16 * i
                _striped(s, lambda o, n, co=co, acc=acc: pltpu.sync_copy(
                    acc.at[pl.ds(o, n)],
                    out_hbm.at[pl.ds(o, n), pl.ds(co, DEG_W)]))


_deg_call = pl.kernel(
    _deg_body,
    out_type=jax.ShapeDtypeStruct((N, 128), jnp.float32),
    mesh=_MESH,
    compiler_params=_SC_PARAMS,
    scratch_types=[
        pltpu.VMEM((DBLK, DEG_W), jnp.float32),
        pltpu.VMEM((DBLK,), jnp.int32),
        pltpu.VMEM((DBLK,), jnp.int32),
        pltpu.VMEM_SHARED((N, DEG_W), jnp.float32),
        pltpu.VMEM_SHARED((N, DEG_W), jnp.float32),
        pltpu.VMEM_SHARED((N, DEG_W), jnp.float32),
        pltpu.SemaphoreType.DMA,
        pltpu.SemaphoreType.DMA,
    ],
)


def _make_edge_body(d, blk, nb):
    R = 4

    def body_fn(y_hbm, e_hbm, zeros_hbm, out_hbm,
                row_all, col0, col1, col2, col3, dat0, dat1, dat2, dat3,
                acc_sh, sl0, sl1, sl2, sl3, ss0, ss1, ss2, ss3):
        cols = (col0, col1, col2, col3)
        dats = (dat0, dat1, dat2, dat3)
        sls = (sl0, sl1, sl2, sl3)
        sss = (ss0, ss1, ss2, ss3)
        c = lax.axis_index("c")
        s = lax.axis_index("s")
        wid = c * NS + s
        ebase = wid * EPW
        pltpu.sync_copy(e_hbm.at[0, pl.ds(ebase, EPW)], row_all)

        @pl.when(c == 0)
        def _():
            _striped(s, lambda o, n: pltpu.sync_copy(
                y_hbm.at[pl.ds(o, n)], acc_sh.at[pl.ds(o, n)]))

        @pl.when(c != 0)
        def _():
            _striped(s, lambda o, n: pltpu.sync_copy(
                zeros_hbm.at[pl.ds(o, n)], acc_sh.at[pl.ds(o, n)]))

        plsc.subcore_barrier()

        def col_desc(j, r):
            return (e_hbm.at[1, _blk(ebase, j, blk)], cols[r], sls[r])

        def g_desc(j, r):
            return (y_hbm.at[row_all.at[_blk(0, j, blk)]], dats[r], sls[r])

        def s_desc(r):
            return (dats[r], acc_sh.at[cols[r]], sss[r])

        def fire_loads(j, r):
            pltpu.async_copy(*col_desc(j, r))
            pltpu.async_copy(*g_desc(j, r))

        fire_loads(0, 0)
        fire_loads(1, 1)

        def step(j, r):
            r2 = (r + 2) % R

            @pl.when(j >= 2)
            def _():
                pltpu.make_async_copy(*s_desc(r2)).wait()

            @pl.when(j + 2 < nb)
            def _():
                fire_loads(j + 2, r2)

            pltpu.make_async_copy(*col_desc(j, r)).wait()
            pltpu.make_async_copy(*g_desc(j, r)).wait()
            pltpu.async_copy(*s_desc(r), add=True)

        def body(j, carry):
            for r in range(R):
                @pl.when(lax.rem(j, R) == r)
                def _(r=r):
                    step(j, r)

            return carry

        lax.fori_loop(0, nb, body, 0)
        pltpu.make_async_copy(*s_desc((nb - 2) % R)).wait()
        pltpu.make_async_copy(*s_desc((nb - 1) % R)).wait()
        plsc.subcore_barrier()
        for cc in range(NC):
            @pl.when(c == cc)
            def _(cc=cc):
                _striped(s, lambda o, n: pltpu.sync_copy(
                    acc_sh.at[pl.ds(o, n)],
                    out_hbm.at[pl.ds(o, n), pl.ds(64 * cc, d)]))

    return body_fn


@functools.cache
def _edge_call(d):
    blk = _EDGE_BLK[d]
    nb = EPW // blk
    return pl.kernel(
        _make_edge_body(d, blk, nb),
        out_type=jax.ShapeDtypeStruct((N, 128), jnp.float32),
        mesh=_MESH,
        compiler_params=_SC_PARAMS,
        scratch_types=(
            [pltpu.VMEM((EPW,), jnp.int32)]
            + [pltpu.VMEM((blk,), jnp.int32) for _ in range(4)]
            + [pltpu.VMEM((blk, d), jnp.float32) for _ in range(4)]
            + [pltpu.VMEM_SHARED((N, d), jnp.float32)]
            + [pltpu.SemaphoreType.DMA for _ in range(8)]
        ),
    )


def _dis(deg_ref, i):
    d = (deg_ref[:, 16 * i:16 * i + 1]
         + deg_ref[:, 64 + 16 * i:64 + 16 * i + 1] + 1.0)
    return lax.rsqrt(d)


def _first_body(x_ref, w_ref, degp_ref, y_ref):
    y_ref[...] = jnp.dot(x_ref[...], w_ref[...],
                         preferred_element_type=jnp.float32) * _dis(degp_ref, 0)


def _make_mid_body(i):
    def body(p_ref, degp_ref, b_ref, w_ref, y_ref):
        d = w_ref.shape[0]
        h = jnp.maximum(
            (p_ref[:, 0:d] + p_ref[:, 64:64 + d]) * _dis(degp_ref, i)
            + b_ref[...], 0.0)
        y_ref[...] = jnp.dot(h, w_ref[...],
                             preferred_element_type=jnp.float32) * _dis(
                                 degp_ref, i + 1)
    return body


def _final_body(p_ref, degp_ref, b_ref, out_ref):
    out_ref[...] = jnp.maximum(
        (p_ref[:, 0:16] + p_ref[:, 64:80]) * _dis(degp_ref, 2)
        + b_ref[...], 0.0)


def _tc(body, out_shape, *args):
    return pl.pallas_call(
        body, out_shape=jax.ShapeDtypeStruct(out_shape, jnp.float32))(*args)


def kernel(features, edge_indexes_1, edge_indexes_3, edge_indexes_9,
           W1, b1, W2, b2, W3, b3):
    ones = jnp.ones((DBLK, DEG_W), jnp.float32)
    zeros64 = jnp.zeros((N, 64), jnp.float32)

    degp = _deg_call(edge_indexes_1, edge_indexes_3, edge_indexes_9,
                     ones, zeros64[:, :DEG_W])

    y1 = _tc(_first_body, (N, 64), features, W1, degp)
    p1 = _edge_call(64)(y1, edge_indexes_1, zeros64)
    y2 = _tc(_make_mid_body(0), (N, 32), p1, degp, b1, W2)
    p2 = _edge_call(32)(y2, edge_indexes_3, zeros64[:, :32])
    y3 = _tc(_make_mid_body(1), (N, 16), p2, degp, b2, W3)
    p3 = _edge_call(16)(y3, edge_indexes_9, zeros64[:, :16])
    h3 = _tc(_final_body, (N, 16), p3, degp, b3)
    return h3


# speedup vs baseline: 1.0123x; 1.0123x over previous
import functools

import jax
import jax.numpy as jnp
from jax import lax
from jax.experimental import pallas as pl
from jax.experimental.pallas import tpu as pltpu
from jax.experimental.pallas import tpu_sc as plsc

N = 10000
E = 320000
NC = 2
NS = 16
NW = NC * NS
EPW = E // NW
RPS = 624
TAIL = N - NS * RPS
DEG_W = 8
DBLK = 1000
DNB = EPW // DBLK
_EDGE_BLK = {64: 200, 32: 400, 16: 1000}

_MESH = plsc.VectorSubcoreMesh(core_axis_name="c", subcore_axis_name="s")
_SC_PARAMS = pltpu.CompilerParams(use_tc_tiling_on_sc=False)


def _striped(s, mk):
    mk(s * RPS, RPS)

    @pl.when(s == NS - 1)
    def _():
        mk(NS * RPS, TAIL)


def _blk(base, j, blk):
    return pl.ds(pl.multiple_of(base + j * blk, 8), blk)


def _deg_body(e1_hbm, e2_hbm, e3_hbm, ones_hbm, zeros_hbm, out_hbm,
              ones_v, col_a, col_b, acc0, acc1, acc2, sem_a, sem_b):
    c = lax.axis_index("c")
    s = lax.axis_index("s")
    wid = c * NS + s
    ebase = wid * EPW
    for acc in (acc0, acc1, acc2):
        _striped(s, lambda o, n, acc=acc: pltpu.sync_copy(
            zeros_hbm.at[pl.ds(o, n)], acc.at[pl.ds(o, n)]))
    pltpu.sync_copy(ones_hbm, ones_v)
    plsc.subcore_barrier()

    for e_hbm, acc in ((e1_hbm, acc0), (e2_hbm, acc1), (e3_hbm, acc2)):
        def start(j, buf, sem, e_hbm=e_hbm):
            pltpu.async_copy(e_hbm.at[1, _blk(ebase, j, DBLK)], buf, sem)

        def proc(j, buf, sem, obuf, osem, e_hbm=e_hbm, acc=acc,
                 start=start):
            @pl.when(j + 1 < DNB)
            def _():
                start(j + 1, obuf, osem)

            pltpu.make_async_copy(
                e_hbm.at[1, _blk(ebase, j, DBLK)], buf, sem).wait()
            pltpu.sync_copy(ones_v, acc.at[buf], add=True)

        start(0, col_a, sem_a)

        def body(j, carry, proc=proc):
            @pl.when(lax.rem(j, 2) == 0)
            def _():
                proc(j, col_a, sem_a, col_b, sem_b)

            @pl.when(lax.rem(j, 2) == 1)
            def _():
                proc(j, col_b, sem_b, col_a, sem_a)

            return carry

        lax.fori_loop(0, DNB, body, 0)
    plsc.subcore_barrier()
    for cc in range(NC):
        @pl.when(c == cc)
        def _(cc=cc):
            for i, acc in enumerate((acc0, acc1, acc2)):
                co = 64 * cc + 16 * i
                _striped(s, lambda o, n, co=co, acc=acc: pltpu.sync_copy(
                    acc.at[pl.ds(o, n)],
                    out_hbm.at[pl.ds(o, n), pl.ds(co, DEG_W)]))


_deg_call = pl.kernel(
    _deg_body,
    out_type=jax.ShapeDtypeStruct((N, 128), jnp.float32),
    mesh=_MESH,
    compiler_params=_SC_PARAMS,
    scratch_types=[
        pltpu.VMEM((DBLK, DEG_W), jnp.float32),
        pltpu.VMEM((DBLK,), jnp.int32),
        pltpu.VMEM((DBLK,), jnp.int32),
        pltpu.VMEM_SHARED((N, DEG_W), jnp.float32),
        pltpu.VMEM_SHARED((N, DEG_W), jnp.float32),
        pltpu.VMEM_SHARED((N, DEG_W), jnp.float32),
        pltpu.SemaphoreType.DMA,
        pltpu.SemaphoreType.DMA,
    ],
)


def _make_edge_body(d, blk, nb):
    R = 4

    def body_fn(y_hbm, e_hbm, zeros_hbm, out_hbm,
                row_all, col0, col1, col2, col3, dat0, dat1, dat2, dat3,
                acc_sh, sl0, sl1, sl2, sl3, ss0, ss1, ss2, ss3):
        cols = (col0, col1, col2, col3)
        dats = (dat0, dat1, dat2, dat3)
        sls = (sl0, sl1, sl2, sl3)
        sss = (ss0, ss1, ss2, ss3)
        c = lax.axis_index("c")
        s = lax.axis_index("s")
        wid = c * NS + s
        ebase = wid * EPW
        pltpu.sync_copy(e_hbm.at[0, pl.ds(ebase, EPW)], row_all)

        @pl.when(c == 0)
        def _():
            _striped(s, lambda o, n: pltpu.sync_copy(
                y_hbm.at[pl.ds(o, n)], acc_sh.at[pl.ds(o, n)]))

        @pl.when(c != 0)
        def _():
            _striped(s, lambda o, n: pltpu.sync_copy(
                zeros_hbm.at[pl.ds(o, n)], acc_sh.at[pl.ds(o, n)]))

        plsc.subcore_barrier()

        def col_desc(j, r):
            return (e_hbm.at[1, _blk(ebase, j, blk)], cols[r], sls[r])

        def g_desc(j, r):
            return (y_hbm.at[row_all.at[_blk(0, j, blk)]], dats[r], sls[r])

        def s_desc(r):
            return (dats[r], acc_sh.at[cols[r]], sss[r])

        def fire_loads(j, r):
            pltpu.async_copy(*col_desc(j, r))
            pltpu.async_copy(*g_desc(j, r))

        fire_loads(0, 0)
        fire_loads(1, 1)

        def step(j, r):
            r2 = (r + 2) % R

            @pl.when(j + 2 < nb)
            def _():
                fire_loads(j + 2, r2)

            pltpu.make_async_copy(*col_desc(j, r)).wait()
            pltpu.make_async_copy(*g_desc(j, r)).wait()
            pltpu.sync_copy(dats[r], acc_sh.at[cols[r]], add=True)

        def body(j, carry):
            for r in range(R):
                @pl.when(lax.rem(j, R) == r)
                def _(r=r):
                    step(j, r)

            return carry

        lax.fori_loop(0, nb, body, 0)
        plsc.subcore_barrier()
        for cc in range(NC):
            @pl.when(c == cc)
            def _(cc=cc):
                _striped(s, lambda o, n: pltpu.sync_copy(
                    acc_sh.at[pl.ds(o, n)],
                    out_hbm.at[pl.ds(o, n), pl.ds(64 * cc, d)]))

    return body_fn


@functools.cache
def _edge_call(d):
    blk = _EDGE_BLK[d]
    nb = EPW // blk
    return pl.kernel(
        _make_edge_body(d, blk, nb),
        out_type=jax.ShapeDtypeStruct((N, 128), jnp.float32),
        mesh=_MESH,
        compiler_params=_SC_PARAMS,
        scratch_types=(
            [pltpu.VMEM((EPW,), jnp.int32)]
            + [pltpu.VMEM((blk,), jnp.int32) for _ in range(4)]
            + [pltpu.VMEM((blk, d), jnp.float32) for _ in range(4)]
            + [pltpu.VMEM_SHARED((N, d), jnp.float32)]
            + [pltpu.SemaphoreType.DMA for _ in range(8)]
        ),
    )


def _dis(deg_ref, i):
    d = (deg_ref[:, 16 * i:16 * i + 1]
         + deg_ref[:, 64 + 16 * i:64 + 16 * i + 1] + 1.0)
    return lax.rsqrt(d)


def _first_body(x_ref, w_ref, degp_ref, y_ref):
    y_ref[...] = jnp.dot(x_ref[...], w_ref[...],
                         preferred_element_type=jnp.float32) * _dis(degp_ref, 0)


def _make_mid_body(i):
    def body(p_ref, degp_ref, b_ref, w_ref, y_ref):
        d = w_ref.shape[0]
        h = jnp.maximum(
            (p_ref[:, 0:d] + p_ref[:, 64:64 + d]) * _dis(degp_ref, i)
            + b_ref[...], 0.0)
        y_ref[...] = jnp.dot(h, w_ref[...],
                             preferred_element_type=jnp.float32) * _dis(
                                 degp_ref, i + 1)
    return body


def _final_body(p_ref, degp_ref, b_ref, out_ref):
    out_ref[...] = jnp.maximum(
        (p_ref[:, 0:16] + p_ref[:, 64:80]) * _dis(degp_ref, 2)
        + b_ref[...], 0.0)


def _tc(body, out_shape, *args):
    return pl.pallas_call(
        body, out_shape=jax.ShapeDtypeStruct(out_shape, jnp.float32))(*args)


def kernel(features, edge_indexes_1, edge_indexes_3, edge_indexes_9,
           W1, b1, W2, b2, W3, b3):
    ones = jnp.ones((DBLK, DEG_W), jnp.float32)
    zeros64 = jnp.zeros((N, 64), jnp.float32)

    degp = _deg_call(edge_indexes_1, edge_indexes_3, edge_indexes_9,
                     ones, zeros64[:, :DEG_W])

    y1 = _tc(_first_body, (N, 64), features, W1, degp)
    p1 = _edge_call(64)(y1, edge_indexes_1, zeros64)
    y2 = _tc(_make_mid_body(0), (N, 32), p1, degp, b1, W2)
    p2 = _edge_call(32)(y2, edge_indexes_3, zeros64[:, :32])
    y3 = _tc(_make_mid_body(1), (N, 16), p2, degp, b2, W3)
    p3 = _edge_call(16)(y3, edge_indexes_9, zeros64[:, :16])
    h3 = _tc(_final_body, (N, 16), p3, degp, b3)
    return h3
